# baseline (speedup 1.0000x reference)
import functools

import jax
import jax.numpy as jnp
from jax.experimental import pallas as pl
from jax.experimental.pallas import tpu as pltpu

_VMEM_LIMIT = 64 * 1024 * 1024
_K = 4


def _pool_kernel(y_ref, lr_hbm, o_ref, lr_bufs, lr_sems, *, k, nsteps):
    bn, bc, H, W = y_ref.shape
    h, w = H // k, W // k
    rows = bn * bc * h

    i = pl.program_id(0)

    @pl.when(i == 0)
    def _prologue():
        pltpu.make_async_copy(lr_hbm.at[pl.ds(0, bn)],
                              lr_bufs.at[0], lr_sems.at[0]).start()
        if nsteps > 1:
            pltpu.make_async_copy(lr_hbm.at[pl.ds(bn, bn)],
                                  lr_bufs.at[1], lr_sems.at[1]).start()

    slot = jax.lax.rem(i, 2)
    pltpu.make_async_copy(lr_bufs.at[slot], lr_bufs.at[slot],
                          lr_sems.at[slot]).wait()

    col = jax.lax.broadcasted_iota(jnp.int32, (W, w), 0) // k
    cell = jax.lax.broadcasted_iota(jnp.int32, (W, w), 1)
    member = (col == cell).astype(jnp.float32)
    m_pool = member * (1.0 / (k * k))

    phases = [y_ref[:, :, r::k, :].reshape(rows, W) for r in range(k)]
    rowsum = phases[0]
    for r in range(1, k):
        rowsum = rowsum + phases[r]
    pooled = jnp.dot(rowsum, m_pool,
                     preferred_element_type=jnp.float32)
    corr = lr_bufs[slot].reshape(rows, w) / pooled
    up = jnp.dot(corr, member.T,
                 preferred_element_type=jnp.float32)
    for r in range(k):
        res = (phases[r] * up).reshape(bn, bc, H // k, W)
        o_ref[:, :, r::k, :] = res.astype(o_ref.dtype)

    @pl.when(i + 2 < nsteps)
    def _prefetch():
        pltpu.make_async_copy(lr_hbm.at[pl.ds((i + 2) * bn, bn)],
                              lr_bufs.at[slot], lr_sems.at[slot]).start()


def kernel(y, lr):
    k = _K
    N, C, H, W = y.shape
    h, w = H // k, W // k

    bn = next(d for d in (4, 2, 1) if N % d == 0)
    nsteps = N // bn
    grid = (nsteps,)

    out = pl.pallas_call(
        functools.partial(_pool_kernel, k=k, nsteps=nsteps),
        out_shape=jax.ShapeDtypeStruct((N, C, H, W), y.dtype),
        grid=grid,
        in_specs=[
            pl.BlockSpec((bn, C, H, W), lambda i: (i, 0, 0, 0)),
            pl.BlockSpec(memory_space=pltpu.MemorySpace.HBM),
        ],
        out_specs=pl.BlockSpec((bn, C, H, W), lambda i: (i, 0, 0, 0)),
        scratch_shapes=[
            pltpu.VMEM((2, bn, C, h, w), jnp.float32),
            pltpu.SemaphoreType.DMA((2,)),
        ],
        compiler_params=pltpu.CompilerParams(
            dimension_semantics=("arbitrary",),
            vmem_limit_bytes=_VMEM_LIMIT,
        ),
    )(y, lr)

    return out


# speedup vs baseline: 1.0393x; 1.0393x over previous
import functools

import jax
import jax.numpy as jnp
from jax.experimental import pallas as pl
from jax.experimental.pallas import tpu as pltpu

_VMEM_LIMIT = 64 * 1024 * 1024
_K = 4


def _pool_kernel(y_ref, lr_ref, o_ref, *, k):
    bn, bc, H, W = y_ref.shape
    h, w = H // k, W // k
    rows = bn * bc * h

    col = jax.lax.broadcasted_iota(jnp.int32, (W, w), 0) // k
    cell = jax.lax.broadcasted_iota(jnp.int32, (W, w), 1)
    member = (col == cell).astype(jnp.float32)
    m_pool = member * (1.0 / (k * k))

    phases = [y_ref[:, :, r::k, :].reshape(rows, W) for r in range(k)]
    rowsum = phases[0]
    for r in range(1, k):
        rowsum = rowsum + phases[r]
    pooled = jnp.dot(rowsum, m_pool,
                     preferred_element_type=jnp.float32)
    i = pl.program_id(0)
    lr = lr_ref[pl.ds(i * bn, bn)]
    corr = lr.reshape(rows, w) / pooled
    up = jnp.dot(corr, member.T,
                 preferred_element_type=jnp.float32)
    for r in range(k):
        res = (phases[r] * up).reshape(bn, bc, H // k, W)
        o_ref[:, :, r::k, :] = res.astype(o_ref.dtype)


def kernel(y, lr):
    k = _K
    N, C, H, W = y.shape
    h, w = H // k, W // k

    bn = next(d for d in (4, 2, 1) if N % d == 0)
    grid = (N // bn,)

    out = pl.pallas_call(
        functools.partial(_pool_kernel, k=k),
        out_shape=jax.ShapeDtypeStruct((N, C, H, W), y.dtype),
        grid=grid,
        in_specs=[
            pl.BlockSpec((bn, C, H, W), lambda i: (i, 0, 0, 0)),
            pl.BlockSpec((N, C, h, w), lambda i: (0, 0, 0, 0)),
        ],
        out_specs=pl.BlockSpec((bn, C, H, W), lambda i: (i, 0, 0, 0)),
        compiler_params=pltpu.CompilerParams(
            dimension_semantics=("arbitrary",),
            vmem_limit_bytes=_VMEM_LIMIT,
        ),
    )(y, lr)

    return out
